# baseline (speedup 1.0000x reference)
import functools

import jax
import jax.numpy as jnp
from jax import lax
from jax.experimental import pallas as pl
from jax.experimental.pallas import tpu as pltpu
from jax.experimental.pallas import tpu_sc as plsc

_BATCH = 4
_NUM_CORES = 2
_NUM_SUBCORES = 16
_NUM_WORKERS = _NUM_CORES * _NUM_SUBCORES


def kernel(x, pe_weight):
    batch = x.shape[0]
    max_len, d_model = pe_weight.shape
    assert batch == _BATCH and max_len % _NUM_WORKERS == 0
    rows_per_worker = max_len // _NUM_WORKERS

    chunk_rows = 64
    n_buf = 2
    assert rows_per_worker % chunk_rows == 0
    n_chunks = rows_per_worker // chunk_rows

    mesh = plsc.VectorSubcoreMesh(core_axis_name="c", subcore_axis_name="s")

    @functools.partial(
        pl.kernel,
        mesh=mesh,
        out_type=jax.ShapeDtypeStruct((batch, max_len, d_model), pe_weight.dtype),
        scratch_types=(
            [pltpu.VMEM((chunk_rows, d_model), pe_weight.dtype)] * n_buf
            + [pltpu.SemaphoreType.DMA] * n_buf
            + [pltpu.SemaphoreType.DMA] * n_buf
        ),
    )
    def _sc_bcast(pe_hbm, out_hbm, *scratch):
        bufs = scratch[:n_buf]
        in_sems = scratch[n_buf : 2 * n_buf]
        out_sems = scratch[2 * n_buf :]
        wid = lax.axis_index("s") * _NUM_CORES + lax.axis_index("c")
        base = wid * rows_per_worker

        def in_copy(i):
            row = base + i * chunk_rows
            return pltpu.make_async_copy(
                pe_hbm.at[pl.ds(row, chunk_rows)], bufs[i % n_buf], in_sems[i % n_buf]
            )

        def out_copies(i):
            row = base + i * chunk_rows
            return [
                pltpu.make_async_copy(
                    bufs[i % n_buf],
                    out_hbm.at[b, pl.ds(row, chunk_rows)],
                    out_sems[i % n_buf],
                )
                for b in range(_BATCH)
            ]

        for i in range(min(n_buf, n_chunks)):
            in_copy(i).start()
        for i in range(n_chunks):
            if i >= n_buf:
                for c in out_copies(i - n_buf):
                    c.wait()
                in_copy(i).start()
            in_copy(i).wait()
            for c in out_copies(i):
                c.start()
        for i in range(max(0, n_chunks - n_buf), n_chunks):
            for c in out_copies(i):
                c.wait()

    return _sc_bcast(pe_weight)
